# Initial kernel scaffold; baseline (speedup 1.0000x reference)
import jax
import jax.numpy as jnp
from jax.experimental import pallas as pl


def kernel(all_logits_2d, tr_logits_3d, tr_img_points, tr_pc2img_idx):
    raise NotImplementedError("write your pallas kernel here")


# speedup vs baseline: 1.2474x; 1.2474x over previous
import functools

import jax
import jax.numpy as jnp
import numpy as np
from jax import lax
from jax.experimental import pallas as pl
from jax.experimental.pallas import tpu as pltpu
from jax.experimental.pallas import tpu_sc as plsc

_B, _C, _H, _W = 4, 19, 512, 512
_NPTS, _N3D = 32768, 131072
_HW = _H * _W
_CW = 32
_CHUNK = 32768
_NCHUNK = _HW // _CHUNK
_TRASH = 64
_NTILES = 16
_PTS_T = _NPTS // _NTILES
_LN19 = float(np.log(19.0))
_INF = float("inf")
_CONF = 0.7


def _softmax_pad_body(x_ref, o_ref):
    x = x_ref[...]
    m = jnp.max(x, axis=1, keepdims=True)
    e = jnp.exp(x - m)
    p = e / jnp.sum(e, axis=1, keepdims=True)
    ones = jnp.ones((x.shape[0], 1), jnp.float32)
    zpad = jnp.zeros((x.shape[0], _CW - _C - 1), jnp.float32)
    o_ref[...] = jnp.concatenate([p, ones, zpad], axis=1)


def _stage_a(tr_logits_3d):
    rb = 4096
    return pl.pallas_call(
        _softmax_pad_body,
        grid=(_N3D // rb,),
        in_specs=[pl.BlockSpec((rb, _C), lambda i: (i, 0))],
        out_specs=pl.BlockSpec((rb, _CW), lambda i: (i, 0)),
        out_shape=jax.ShapeDtypeStruct((_N3D, _CW), jnp.float32),
    )(tr_logits_3d)


def _sc_body(sm3_hbm, idx_hbm, pts_hbm, acc_hbm,
             idx_v, h_v, pts_v, gidx_v, swidx_v, gidx2_v, sidx2_v, rows_v,
             zblk_v, shared, sem):
    core = lax.axis_index("c")
    tile = lax.axis_index("s")
    iota16 = lax.iota(jnp.int32, 16)
    z16 = jnp.zeros((16,), jnp.float32)
    trash16 = _CHUNK + iota16
    zero_i16 = jnp.zeros((16,), jnp.int32)

    def _zrow(r, _):
        zblk_v[r, pl.ds(0, 16)] = z16
        zblk_v[r, pl.ds(16, 16)] = z16
        return 0
    lax.fori_loop(0, 512, _zrow, 0)

    stripe = tile * 2052

    for ii in range(2):
        img = core * 2 + ii
        pltpu.sync_copy(idx_hbm.at[img, tile], idx_v)
        pltpu.sync_copy(pts_hbm.at[img, tile], pts_v)

        def _mkh(g, _):
            ii16 = g * 16 + iota16
            r = plsc.load_gather(pts_v, [ii16 * 2])
            c = plsc.load_gather(pts_v, [ii16 * 2 + 1])
            h_v[pl.ds(g * 16, 16)] = r * _W + c
            return 0
        lax.fori_loop(0, _PTS_T // 16, _mkh, 0)

        for ch in range(_NCHUNK):
            for k in range(4):
                pltpu.sync_copy(zblk_v,
                                shared.at[pl.ds(stripe + k * 512, 512)])
            pltpu.sync_copy(zblk_v.at[pl.ds(0, 4)],
                            shared.at[pl.ds(stripe + 2048, 4)])

            def _pf(r, _):
                gidx_v[pl.ds(r * 16, 16)] = zero_i16
                swidx_v[pl.ds(r * 16, 16)] = trash16
                return 0
            lax.fori_loop(0, 160, _pf, 0)

            cb = ch * _CHUNK

            def _cmp(g, off):
                hh = h_v[pl.ds(g * 16, 16)]
                local = hh - cb
                inb = (local >= 0) & (local < _CHUNK)
                idxg = idx_v[pl.ds(g * 16, 16)]
                inb_i = inb.astype(jnp.int32)
                excl = plsc.cumsum(inb_i) - inb_i
                pos = jnp.where(inb, off + excl, 2544 + iota16)
                plsc.store_scatter(gidx_v, [pos], idxg)
                plsc.store_scatter(swidx_v, [pos], local)
                return off + jnp.sum(inb_i)
            cnt = lax.fori_loop(0, _PTS_T // 16, _cmp, jnp.int32(0))
            plsc.subcore_barrier()

            def _blk(b, _):
                base = pl.multiple_of(b * 128, 128)
                for k in range(8):
                    gidx2_v[0, pl.ds(k * 16, 16)] = gidx_v[pl.ds(base + k * 16, 16)]
                    sidx2_v[0, pl.ds(k * 16, 16)] = swidx_v[pl.ds(base + k * 16, 16)]
                pltpu.async_copy(sm3_hbm.at[gidx2_v.at[0]], rows_v, sem).wait()
                pltpu.sync_copy(rows_v, shared.at[sidx2_v.at[0]], add=True)
                return 0
            lax.fori_loop(0, (cnt + 127) // 128, _blk, 0)
            plsc.subcore_barrier()

            orows = _CHUNK // _NTILES
            pltpu.sync_copy(shared.at[pl.ds(tile * orows, orows)],
                            acc_hbm.at[img, ch, pl.ds(tile * orows, orows)])
            plsc.subcore_barrier()


def _stage_c(sm3, idx3, pts3):
    mesh = plsc.VectorSubcoreMesh(core_axis_name="c", subcore_axis_name="s")
    fn = functools.partial(
        pl.kernel,
        mesh=mesh,
        compiler_params=pltpu.CompilerParams(
            use_tc_tiling_on_sc=False, needs_layout_passes=False),
        out_type=jax.ShapeDtypeStruct((_B, _NCHUNK, _CHUNK, _CW), jnp.float32),
        scratch_types=[
            pltpu.VMEM((_PTS_T,), jnp.int32),
            pltpu.VMEM((_PTS_T,), jnp.int32),
            pltpu.VMEM((2 * _PTS_T,), jnp.int32),
            pltpu.VMEM((2560,), jnp.int32),
            pltpu.VMEM((2560,), jnp.int32),
            pltpu.VMEM((1, 128), jnp.int32),
            pltpu.VMEM((1, 128), jnp.int32),
            pltpu.VMEM((128, _CW), jnp.float32),
            pltpu.VMEM((512, _CW), jnp.float32),
            pltpu.VMEM_SHARED((_CHUNK + _TRASH, _CW), jnp.float32),
            pltpu.SemaphoreType.DMA,
        ],
    )(_sc_body)
    return fn(sm3, idx3, pts3)


def _dense_body(acc_ref, img_ref, o_ref):
    xt = acc_ref[0].T
    cnt = xt[19]
    valid = cnt > 0.0
    rsafe = 1.0 / jnp.where(valid, cnt, 1.0)

    x = img_ref[0]
    m = jnp.max(x, axis=0)
    xm = x - m[None, :]
    e = jnp.exp(xm)
    ssum = jnp.sum(e, axis=0)
    lns = jnp.log(ssum)
    rssum = 1.0 / ssum

    s_dd = jnp.zeros_like(cnt)
    s_dxm = jnp.zeros_like(cnt)
    s_pxm = jnp.zeros_like(cnt)
    s_id = jnp.zeros_like(cnt)
    sumds = jnp.zeros_like(cnt)
    sump = jnp.zeros_like(cnt)
    for c in range(_C):
        ds_c = xt[c] * rsafe
        lds_c = jnp.log(jnp.where(ds_c > 0.0, ds_c, 1.0))
        xm_c = xm[c]
        p_c = e[c] * rssum
        s_dd += ds_c * lds_c
        s_dxm += ds_c * xm_c
        s_pxm += p_c * xm_c
        s_id += p_c * lds_c
        sumds += ds_c
        sump += p_c

    s_ii = s_pxm - sump * lns
    s_di = s_dxm - sumds * lns
    ety3 = -s_dd / _LN19
    ety2 = -s_ii / _LN19
    e3s = jnp.where(valid, ety3, 1.0)
    e2s = jnp.where(valid, ety2, 1.0)
    rv2 = 1.0 / e2s + 1e-30
    rv3 = 1.0 / e3s + 1e-30
    w2 = rv2 / (rv2 + rv3)
    w3 = rv3 / (rv2 + rv3)
    kl23 = w3 * (s_dd - s_di)
    kl32 = w2 * (s_ii - s_id)

    o_ref[0, 0, :] = jnp.where(valid, ety3, _INF)
    o_ref[0, 1, :] = jnp.where(valid, ety2, _INF)
    o_ref[0, 2, :] = jnp.where(valid, kl23, 0.0)
    o_ref[0, 3, :] = jnp.where(valid, kl32, 0.0)


def _stage_d(acc4, img3):
    rb = 8192
    return pl.pallas_call(
        _dense_body,
        grid=(_B, _HW // rb),
        in_specs=[
            pl.BlockSpec((1, rb, _CW), lambda i, b: (i, b, 0)),
            pl.BlockSpec((1, _C, rb), lambda i, b: (i, 0, b)),
        ],
        out_specs=pl.BlockSpec((1, 4, rb), lambda i, b: (i, 0, b)),
        out_shape=jax.ShapeDtypeStruct((_B, 4, _HW), jnp.float32),
    )(acc4, img3)


def _order_stat(u, k):
    def body(it, xc):
        t = xc | jnp.left_shift(jnp.int32(1), jnp.int32(30 - it))
        cnt_lt = jnp.sum((u < t).astype(jnp.int32))
        return jnp.where(cnt_lt <= k, t, xc)
    return lax.fori_loop(0, 31, body, jnp.int32(0))


def _thred(ety, li, hi, lw_, hw_):
    u = lax.bitcast_convert_type(ety, jnp.int32)
    x_li = _order_stat(u, li)
    cle = jnp.sum((u <= x_li).astype(jnp.int32))
    gt_min = jnp.min(jnp.where(u > x_li, u, jnp.int32(0x7F800000)))
    x_hi = jnp.where(cle >= hi + 1, x_li, gt_min)
    f_li = lax.bitcast_convert_type(x_li, jnp.float32)
    f_hi = lax.bitcast_convert_type(x_hi, jnp.float32)
    return f_li * lw_ + f_hi * hw_


def _final_body(d_ref, o_ref):
    e3 = d_ref[0, 0, :]
    e2 = d_ref[0, 1, :]
    k23 = d_ref[0, 2, :]
    k32 = d_ref[0, 3, :]
    nun = jnp.sum((e3 < _INF).astype(jnp.int32))
    nf = nun.astype(jnp.float32)
    q = jnp.float32(_CONF) * (nf - 1.0)
    low = jnp.floor(q)
    high = jnp.ceil(q)
    hw_ = q - low
    lw_ = 1.0 - hw_
    li = low.astype(jnp.int32)
    hi = high.astype(jnp.int32)
    thred3 = _thred(e3, li, hi, lw_, hw_)
    thred2 = _thred(e2, li, hi, lw_, hw_)
    mask3 = e3 < thred3
    mask2 = e2 < thred2
    m23 = (jnp.sum(jnp.where(mask3, k23, 0.0))
           / jnp.sum(mask3.astype(jnp.float32)))
    m32 = (jnp.sum(jnp.where(mask2, k32, 0.0))
           / jnp.sum(mask2.astype(jnp.float32)))
    o_ref[...] = jnp.zeros((1, 8, 128), jnp.float32) + (m23 + m32)


def _stage_e(dense):
    out = pl.pallas_call(
        _final_body,
        grid=(_B,),
        in_specs=[pl.BlockSpec((1, 4, _HW), lambda i: (i, 0, 0))],
        out_specs=pl.BlockSpec((1, 8, 128), lambda i: (i, 0, 0)),
        out_shape=jax.ShapeDtypeStruct((_B, 8, 128), jnp.float32),
    )(dense)
    return jnp.sum(out[:, 0, 0]) / _B


def kernel(all_logits_2d, tr_logits_3d, tr_img_points, tr_pc2img_idx):
    sm3 = _stage_a(tr_logits_3d)
    idx3 = tr_pc2img_idx.astype(jnp.int32).reshape(_B, _NTILES, _PTS_T)
    pts3 = tr_img_points.astype(jnp.int32).reshape(_B, _NTILES, 2 * _PTS_T)
    acc = _stage_c(sm3, idx3, pts3)
    acc4 = acc.reshape(_B, _HW, _CW)
    dense = _stage_d(acc4, all_logits_2d.reshape(_B, _C, _HW))
    return _stage_e(dense)


# speedup vs baseline: 1.6446x; 1.3184x over previous
import functools

import jax
import jax.numpy as jnp
import numpy as np
from jax import lax
from jax.experimental import pallas as pl
from jax.experimental.pallas import tpu as pltpu
from jax.experimental.pallas import tpu_sc as plsc

_B, _C, _H, _W = 4, 19, 512, 512
_NPTS, _N3D = 32768, 131072
_HW = _H * _W
_CW = 32
_ACW = 20
_CHUNK = 32768
_NCHUNK = _HW // _CHUNK
_TRASH = 64
_NTILES = 16
_PTS_T = _NPTS // _NTILES
_LN19 = float(np.log(19.0))
_INF = float("inf")
_CONF = 0.7


def _softmax_pad_body(x_ref, o_ref):
    x = x_ref[...]
    m = jnp.max(x, axis=1, keepdims=True)
    e = jnp.exp(x - m)
    p = e / jnp.sum(e, axis=1, keepdims=True)
    ones = jnp.ones((x.shape[0], 1), jnp.float32)
    zpad = jnp.zeros((x.shape[0], _CW - _C - 1), jnp.float32)
    o_ref[...] = jnp.concatenate([p, ones, zpad], axis=1)


def _stage_a(tr_logits_3d):
    rb = 4096
    return pl.pallas_call(
        _softmax_pad_body,
        grid=(_N3D // rb,),
        in_specs=[pl.BlockSpec((rb, _C), lambda i: (i, 0))],
        out_specs=pl.BlockSpec((rb, _CW), lambda i: (i, 0)),
        out_shape=jax.ShapeDtypeStruct((_N3D, _CW), jnp.float32),
    )(tr_logits_3d)


def _sc_body(sm3_hbm, idx_hbm, pts_hbm, acc_hbm,
             idx_v, h_v, pts_v, gidx_v, swidx_v, gidx2_v, sidx2_v,
             rows_v, zblk_v, shared, sem):
    core = lax.axis_index("c")
    tile = lax.axis_index("s")
    iota16 = lax.iota(jnp.int32, 16)
    z16 = jnp.zeros((16,), jnp.float32)
    trash16 = _CHUNK + iota16
    zero_i16 = jnp.zeros((16,), jnp.int32)

    def _zrow(r, _):
        zblk_v[r, pl.ds(0, 16)] = z16
        zblk_v[r, pl.ds(16, 16)] = z16
        return 0
    lax.fori_loop(0, 512, _zrow, 0)

    for k in range(4):
        pltpu.sync_copy(zblk_v, shared.at[pl.ds(tile * 2052 + k * 512, 512)])
    pltpu.sync_copy(zblk_v.at[pl.ds(0, 4)],
                    shared.at[pl.ds(tile * 2052 + 2048, 4)])
    plsc.subcore_barrier()

    for ii in range(2):
        img = core * 2 + ii
        pltpu.sync_copy(idx_hbm.at[img, tile], idx_v)
        pltpu.sync_copy(pts_hbm.at[img, tile], pts_v)

        def _mkh(g, _):
            ii16 = g * 16 + iota16
            r = plsc.load_gather(pts_v, [ii16 * 2])
            c = plsc.load_gather(pts_v, [ii16 * 2 + 1])
            h_v[pl.ds(g * 16, 16)] = r * _W + c
            return 0
        lax.fori_loop(0, _PTS_T // 16, _mkh, 0)

        for ch in range(_NCHUNK):
            def _pf(r, _):
                gidx_v[pl.ds(r * 16, 16)] = zero_i16
                swidx_v[pl.ds(r * 16, 16)] = trash16
                return 0
            lax.fori_loop(0, 160, _pf, 0)

            cb = ch * _CHUNK

            def _cmp(g, off):
                hh = h_v[pl.ds(g * 16, 16)]
                local = hh - cb
                inb = (local >= 0) & (local < _CHUNK)
                idxg = idx_v[pl.ds(g * 16, 16)]
                inb_i = inb.astype(jnp.int32)
                excl = plsc.cumsum(inb_i) - inb_i
                pos = jnp.where(inb, off + excl, 2544 + iota16)
                plsc.store_scatter(gidx_v, [pos], idxg)
                plsc.store_scatter(swidx_v, [pos], local)
                return off + jnp.sum(inb_i)
            cnt = lax.fori_loop(0, _PTS_T // 16, _cmp, jnp.int32(0))

            def _blk(b, _):
                base = pl.multiple_of(b * 128, 128)
                for k in range(8):
                    gidx2_v[0, pl.ds(k * 16, 16)] = gidx_v[pl.ds(base + k * 16, 16)]
                    sidx2_v[0, pl.ds(k * 16, 16)] = swidx_v[pl.ds(base + k * 16, 16)]
                pltpu.async_copy(sm3_hbm.at[gidx2_v.at[0]], rows_v, sem).wait()
                pltpu.sync_copy(rows_v, shared.at[sidx2_v.at[0]], add=True)
                return 0
            lax.fori_loop(0, (cnt + 127) // 128, _blk, 0)
            plsc.subcore_barrier()

            pltpu.sync_copy(shared.at[pl.ds(tile * 2048, 2048)],
                            acc_hbm.at[img, ch, pl.ds(tile * 2048, 2048)])
            for k in range(4):
                pltpu.sync_copy(zblk_v,
                                shared.at[pl.ds(tile * 2048 + k * 512, 512)])
            pltpu.sync_copy(zblk_v.at[pl.ds(0, 4)],
                            shared.at[pl.ds(_CHUNK + tile * 4, 4)])
            plsc.subcore_barrier()


def _stage_c(sm3, idx3, pts3):
    mesh = plsc.VectorSubcoreMesh(core_axis_name="c", subcore_axis_name="s")
    fn = functools.partial(
        pl.kernel,
        mesh=mesh,
        compiler_params=pltpu.CompilerParams(
            use_tc_tiling_on_sc=False, needs_layout_passes=False),
        out_type=jax.ShapeDtypeStruct((_B, _NCHUNK, _CHUNK, _CW), jnp.float32),
        scratch_types=[
            pltpu.VMEM((_PTS_T,), jnp.int32),
            pltpu.VMEM((_PTS_T,), jnp.int32),
            pltpu.VMEM((2 * _PTS_T,), jnp.int32),
            pltpu.VMEM((2560,), jnp.int32),
            pltpu.VMEM((2560,), jnp.int32),
            pltpu.VMEM((1, 128), jnp.int32),
            pltpu.VMEM((1, 128), jnp.int32),
            pltpu.VMEM((128, _CW), jnp.float32),
            pltpu.VMEM((512, _CW), jnp.float32),
            pltpu.VMEM_SHARED((_CHUNK + _TRASH, _CW), jnp.float32),
            pltpu.SemaphoreType.DMA,
        ],
    )(_sc_body)
    return fn(sm3, idx3, pts3)


def _dense_body(acc_ref, img_ref, o_ref):
    at = acc_ref[0]
    cnt = at[19]
    valid = cnt > 0.0
    rsafe = 1.0 / jnp.where(valid, cnt, 1.0)

    x = img_ref[0]
    m = jnp.max(x, axis=0)
    xm = x - m[None]
    e = jnp.exp(xm)
    ssum = jnp.sum(e, axis=0)
    lns = jnp.log(ssum)
    rssum = 1.0 / ssum

    s_dd = jnp.zeros_like(cnt)
    s_dxm = jnp.zeros_like(cnt)
    s_pxm = jnp.zeros_like(cnt)
    s_id = jnp.zeros_like(cnt)
    sumds = jnp.zeros_like(cnt)
    sump = jnp.zeros_like(cnt)
    for c in range(_C):
        ds_c = at[c] * rsafe
        lds_c = jnp.log(jnp.where(ds_c > 0.0, ds_c, 1.0))
        xm_c = xm[c]
        p_c = e[c] * rssum
        s_dd += ds_c * lds_c
        s_dxm += ds_c * xm_c
        s_pxm += p_c * xm_c
        s_id += p_c * lds_c
        sumds += ds_c
        sump += p_c

    s_ii = s_pxm - sump * lns
    s_di = s_dxm - sumds * lns
    ety3 = -s_dd / _LN19
    ety2 = -s_ii / _LN19
    e3s = jnp.where(valid, ety3, 1.0)
    e2s = jnp.where(valid, ety2, 1.0)
    rv2 = 1.0 / e2s + 1e-30
    rv3 = 1.0 / e3s + 1e-30
    w2 = rv2 / (rv2 + rv3)
    w3 = rv3 / (rv2 + rv3)
    kl23 = w3 * (s_dd - s_di)
    kl32 = w2 * (s_ii - s_id)

    o_ref[0, 0] = jnp.where(valid, ety3, _INF)
    o_ref[0, 1] = jnp.where(valid, ety2, _INF)
    o_ref[0, 2] = jnp.where(valid, kl23, 0.0)
    o_ref[0, 3] = jnp.where(valid, kl32, 0.0)


def _stage_d(acc_t4, img4):
    nb = _HW // 128 // 64
    return pl.pallas_call(
        _dense_body,
        grid=(_B, nb),
        in_specs=[
            pl.BlockSpec((1, _ACW, 64, 128), lambda i, b: (i, 0, b, 0)),
            pl.BlockSpec((1, _C, 64, 128), lambda i, b: (i, 0, b, 0)),
        ],
        out_specs=pl.BlockSpec((1, 4, 64, 128), lambda i, b: (i, 0, b, 0)),
        out_shape=jax.ShapeDtypeStruct((_B, 4, _HW // 128, 128), jnp.float32),
    )(acc_t4, img4)


def _order_stat(u, k):
    def body(it, xc):
        t = xc | jnp.left_shift(jnp.int32(1), jnp.int32(30 - it))
        cnt_lt = jnp.sum((u < t).astype(jnp.int32))
        return jnp.where(cnt_lt <= k, t, xc)
    return lax.fori_loop(0, 31, body, jnp.int32(0))


def _thred(ety, li, hi, lw_, hw_):
    u = lax.bitcast_convert_type(ety, jnp.int32)
    x_li = _order_stat(u, li)
    cle = jnp.sum((u <= x_li).astype(jnp.int32))
    gt_min = jnp.min(jnp.where(u > x_li, u, jnp.int32(0x7F800000)))
    x_hi = jnp.where(cle >= hi + 1, x_li, gt_min)
    f_li = lax.bitcast_convert_type(x_li, jnp.float32)
    f_hi = lax.bitcast_convert_type(x_hi, jnp.float32)
    return f_li * lw_ + f_hi * hw_


def _final_body(d_ref, o_ref):
    e3 = d_ref[0, 0]
    e2 = d_ref[0, 1]
    k23 = d_ref[0, 2]
    k32 = d_ref[0, 3]
    nun = jnp.sum((e3 < _INF).astype(jnp.int32))
    nf = nun.astype(jnp.float32)
    q = jnp.float32(_CONF) * (nf - 1.0)
    low = jnp.floor(q)
    high = jnp.ceil(q)
    hw_ = q - low
    lw_ = 1.0 - hw_
    li = low.astype(jnp.int32)
    hi = high.astype(jnp.int32)
    thred3 = _thred(e3, li, hi, lw_, hw_)
    thred2 = _thred(e2, li, hi, lw_, hw_)
    mask3 = e3 < thred3
    mask2 = e2 < thred2
    m23 = (jnp.sum(jnp.where(mask3, k23, 0.0))
           / jnp.sum(mask3.astype(jnp.float32)))
    m32 = (jnp.sum(jnp.where(mask2, k32, 0.0))
           / jnp.sum(mask2.astype(jnp.float32)))
    o_ref[...] = jnp.zeros((1, 8, 128), jnp.float32) + (m23 + m32)


def _stage_e(dense):
    out = pl.pallas_call(
        _final_body,
        grid=(_B,),
        in_specs=[pl.BlockSpec((1, 4, _HW // 128, 128), lambda i: (i, 0, 0, 0))],
        out_specs=pl.BlockSpec((1, 8, 128), lambda i: (i, 0, 0)),
        out_shape=jax.ShapeDtypeStruct((_B, 8, 128), jnp.float32),
    )(dense)
    return jnp.sum(out[:, 0, 0]) / _B


def kernel(all_logits_2d, tr_logits_3d, tr_img_points, tr_pc2img_idx):
    sm3 = _stage_a(tr_logits_3d)
    idx3 = tr_pc2img_idx.astype(jnp.int32).reshape(_B, _NTILES, _PTS_T)
    pts3 = tr_img_points.astype(jnp.int32).reshape(_B, _NTILES, 2 * _PTS_T)
    acc = _stage_c(sm3, idx3, pts3)
    acc_t = jnp.transpose(acc.reshape(_B, _HW, _CW)[:, :, :_ACW], (0, 2, 1))
    dense = _stage_d(acc_t.reshape(_B, _ACW, _HW // 128, 128),
                     all_logits_2d.reshape(_B, _C, _HW // 128, 128))
    return _stage_e(dense)
